# Initial kernel scaffold; baseline (speedup 1.0000x reference)
import jax
import jax.numpy as jnp
from jax.experimental import pallas as pl


def kernel(predictions, targets):
    raise NotImplementedError("write your pallas kernel here")


# speedup vs baseline: 1.0065x; 1.0065x over previous
import functools

import jax
import jax.numpy as jnp
from jax import lax
from jax.experimental import pallas as pl
from jax.experimental.pallas import tpu as pltpu

_N = 16384
_C = 10
_BI = 512
_BJ = 2048
_LS = 0.1
_LAM = 0.5


def _prep_body(x_ref, s_ref):
    x = x_ref[...]
    m = jnp.max(x, axis=0, keepdims=True)
    e = jnp.exp(x - m)
    tot = jnp.sum(e, axis=0, keepdims=True)
    s_ref[...] = (x - m) - jnp.log(tot)


def _count_body(s_ref, t_ref, stat_ref):
    i = pl.program_id(0)

    t = t_ref[pl.ds(i * _BI, _BI)]
    s_i = s_ref[:, pl.ds(i * _BI, _BI)]

    cls_col = lax.broadcasted_iota(jnp.int32, (_C, _BI), 0)
    ht = (cls_col == t[None, :]).astype(jnp.float32)
    cls_row = lax.broadcasted_iota(jnp.int32, (_BI, _C), 1)
    h = (cls_row == t[:, None]).astype(jnp.float32)

    o = jnp.sum(ht * s_i, axis=0)
    colsum = jnp.sum(s_i, axis=0)
    ce_part = jnp.sum(-((1.0 - _LS) * o + (_LS / _C) * colsum))

    def body(j, cnt):
        s_j = s_ref[:, pl.ds(j * _BJ, _BJ)]
        g = lax.dot_general(h, s_j, (((1,), (0,)), ((), ())),
                            preferred_element_type=jnp.float32)
        m = (g < o[:, None]).astype(jnp.float32)
        return cnt + jnp.sum(m, axis=1)

    cnt = lax.fori_loop(0, _N // _BJ, body, jnp.zeros((_BI,), jnp.float32))

    lane = lax.broadcasted_iota(jnp.int32, (_BI, 128), 1)
    m128 = (lane == t[:, None]).astype(jnp.float32)
    r128 = jnp.sum(m128 * cnt[:, None], axis=0)
    p128 = jnp.sum(m128, axis=0)

    row = lax.broadcasted_iota(jnp.int32, (8, 128), 0)
    upd = jnp.where(row == 0, r128[None, :],
                    jnp.where(row == 1, p128[None, :],
                              jnp.where(row == 2, ce_part, 0.0)))

    @pl.when(i == 0)
    def _():
        stat_ref[...] = jnp.zeros((8, 128), jnp.float32)

    stat_ref[...] += upd


def _final_body(stat_ref, out_ref):
    r = stat_ref[0, :]
    p = stat_ref[1, :]
    ce_sum = stat_ref[2, 0]
    n = _N - p
    u = r - p * (p - 1.0) * 0.5
    denom = jnp.maximum(p, 1.0) * jnp.maximum(n, 1.0)
    auc = u / denom
    lane = lax.iota(jnp.int32, 128)
    auc = jnp.where(lane < _C, auc, 0.0)
    pauc = jnp.sum(auc) / _C
    ce = ce_sum / _N
    loss = (1.0 - _LAM) * ce + _LAM * (1.0 - pauc * pauc)
    out_ref[0, 0] = loss


@jax.jit
def kernel(predictions, targets):
    pred_t = predictions.T

    s = pl.pallas_call(
        _prep_body,
        grid=(_N // _BJ,),
        in_specs=[pl.BlockSpec((_C, _BJ), lambda j: (0, j))],
        out_specs=pl.BlockSpec((_C, _BJ), lambda j: (0, j)),
        out_shape=jax.ShapeDtypeStruct((_C, _N), jnp.float32),
    )(pred_t)

    stat = pl.pallas_call(
        _count_body,
        grid=(_N // _BI,),
        in_specs=[
            pl.BlockSpec((_C, _N), lambda i: (0, 0)),
            pl.BlockSpec((_N,), lambda i: (0,)),
        ],
        out_specs=pl.BlockSpec((8, 128), lambda i: (0, 0)),
        out_shape=jax.ShapeDtypeStruct((8, 128), jnp.float32),
    )(s, targets)

    out = pl.pallas_call(
        _final_body,
        in_specs=[pl.BlockSpec((8, 128), lambda: (0, 0))],
        out_specs=pl.BlockSpec((1, 1), lambda: (0, 0), memory_space=pltpu.SMEM),
        out_shape=jax.ShapeDtypeStruct((1, 1), jnp.float32),
    )(stat)

    return out[0, 0]
